# baseline (speedup 1.0000x reference)
import functools

import jax
import jax.numpy as jnp
from jax import lax
from jax.experimental import pallas as pl
from jax.experimental.pallas import tpu as pltpu
from jax.experimental.pallas import tpu_sc as plsc

_B, _L, _D = 4096, 50, 128

try:
    _info = plsc.get_sparse_core_info()
    _NC, _NS = _info.num_cores, _info.num_subcores
except Exception:
    _NC, _NS = 2, 16
_NW = _NC * _NS
_BW = _B // _NW
_CB = 2
_NCH = _BW // _CB

_mesh = plsc.VectorSubcoreMesh(core_axis_name="c", subcore_axis_name="s")


@functools.partial(
    pl.kernel,
    out_type=jax.ShapeDtypeStruct((_B, _L, _D), jnp.float32),
    mesh=_mesh,
    compiler_params=pltpu.CompilerParams(use_tc_tiling_on_sc=True),
    scratch_types=[
        pltpu.VMEM((_NCH, 128), jnp.int32),
        pltpu.VMEM((2, _CB * _L, _D), jnp.float32),
        pltpu.SemaphoreType.DMA,
        pltpu.SemaphoreType.DMA,
        pltpu.SemaphoreType.DMA,
        pltpu.SemaphoreType.DMA,
    ],
)
def _emb_lookup(idx_hbm, table_hbm, out_hbm, idx_flat, rows_v, g0, g1, s0, s1):
    wid = lax.axis_index("s") * _NC + lax.axis_index("c")
    b0 = wid * _BW
    gsem = (g0, g1)
    ssem = (s0, s1)
    pltpu.sync_copy(idx_hbm.at[wid], idx_flat)

    def _gather(c, b):
        return pltpu.make_async_copy(
            table_hbm.at[idx_flat.at[c, pl.ds(0, _CB * _L)]],
            rows_v.at[b], gsem[b])

    def _scatter(c, b):
        return pltpu.make_async_copy(
            rows_v.at[b].reshape(_CB, _L, _D),
            out_hbm.at[pl.ds(b0 + c * _CB, _CB)],
            ssem[b])

    _gather(0, 0).start()

    @pl.loop(0, _NCH, step=2)
    def _body(g):
        for b in range(2):
            c = g + b
            bn = (b + 1) % 2
            _gather(c, b).wait()
            _scatter(c, b).start()

            @pl.when(c > 0)
            def _():
                _scatter(c - 1, bn).wait()

            @pl.when(c + 1 < _NCH)
            def _():
                _gather(c + 1, bn).start()

    _scatter(_NCH - 1, (_NCH - 1) % 2).wait()


def kernel(x, embedding_table):
    idx = x.astype(jnp.int32).reshape(_NW, _NCH, _CB * _L)
    idx = jnp.pad(idx, ((0, 0), (0, 0), (0, 128 - _CB * _L)))
    return _emb_lookup(idx, embedding_table)


# speedup vs baseline: 2.1123x; 2.1123x over previous
import functools

import jax
import jax.numpy as jnp
from jax import lax
from jax.experimental import pallas as pl
from jax.experimental.pallas import tpu as pltpu
from jax.experimental.pallas import tpu_sc as plsc

_B, _L, _D = 4096, 50, 128

try:
    _info = plsc.get_sparse_core_info()
    _NC, _NS = _info.num_cores, _info.num_subcores
except Exception:
    _NC, _NS = 2, 16
_NW = _NC * _NS
_BW = _B // _NW

_mesh = plsc.VectorSubcoreMesh(core_axis_name="c", subcore_axis_name="s")


@functools.partial(
    pl.kernel,
    out_type=jax.ShapeDtypeStruct((_L, _B, _D), jnp.float32),
    mesh=_mesh,
    scratch_types=[
        pltpu.VMEM((_L, _BW), jnp.int32),
        pltpu.VMEM((6, _BW, _D), jnp.float32),
        [pltpu.SemaphoreType.DMA] * 6,
        [pltpu.SemaphoreType.DMA] * 6,
    ],
)
def _emb_lookup(idx_hbm, table_hbm, out_hbm, idx_v, rows_v, gsem, ssem):
    wid = lax.axis_index("s") * _NC + lax.axis_index("c")
    b0 = wid * _BW
    pltpu.sync_copy(idx_hbm.at[:, pl.ds(b0, _BW)], idx_v)

    def _gather(c, b):
        return pltpu.make_async_copy(
            table_hbm.at[idx_v.at[c]], rows_v.at[b], gsem[b])

    def _scatter(c, b):
        return pltpu.make_async_copy(
            rows_v.at[b], out_hbm.at[c, pl.ds(b0, _BW)], ssem[b])

    def _step(c, b, py_tail=False):
        _gather(c, b).wait()
        _scatter(c, b).start()
        if py_tail:
            if c >= 3:
                _scatter(c - 3, (c - 3) % 6).wait()
            if c + 3 < _L:
                _gather(c + 3, (c + 3) % 6).start()
        else:
            @pl.when(c >= 3)
            def _():
                _scatter(c - 3, (b - 3) % 6).wait()

            @pl.when(c + 3 < _L)
            def _():
                _gather(c + 3, (b + 3) % 6).start()

    for c in range(3):
        _gather(c, c).start()

    _MAIN = (_L // 6) * 6

    @pl.loop(0, _MAIN, step=6)
    def _body(g):
        for b in range(6):
            _step(g + b, b)

    for c in range(_MAIN, _L):
        _step(c, c % 6, py_tail=True)

    for c in range(_L - 3, _L):
        _scatter(c, c % 6).wait()


def kernel(x, embedding_table):
    idx_t = jnp.swapaxes(x.astype(jnp.int32), 0, 1)
    out_t = _emb_lookup(idx_t, embedding_table)
    return jnp.swapaxes(out_t, 0, 1)
